# Initial kernel scaffold; baseline (speedup 1.0000x reference)
import jax
import jax.numpy as jnp
from jax.experimental import pallas as pl


def kernel(features, W, b, a, edge_index, nodes, ind):
    raise NotImplementedError("write your pallas kernel here")


# speedup vs baseline: 6.3382x; 6.3382x over previous
import functools

import jax
import jax.numpy as jnp
from jax import lax
from jax.experimental import pallas as pl
from jax.experimental.pallas import tpu as pltpu
from jax.experimental.pallas import tpu_sc as plsc

N = 10000
E = 320000
D = 128
DEXT = 144
SLOPE = 0.1

NC, NS, L = 2, 16, 16
NW = NC * NS
EPW = E // NW
K = 80
NCHUNK = EPW // K
NRCHUNK = N // K


def _prep_body(f_ref, w_ref, b_ref, a_ref, emb_ref, al_ref):
    emb = jnp.dot(f_ref[...], w_ref[...], preferred_element_type=jnp.float32)
    emb = emb + b_ref[...]
    al_ref[...] = jnp.dot(emb, a_ref[...], preferred_element_type=jnp.float32)
    ones = jnp.ones((N, 1), jnp.float32)
    pad = jnp.zeros((N, DEXT - D - 1), jnp.float32)
    emb_ref[...] = jnp.concatenate([emb, ones, pad], axis=1)


_prep = pl.pallas_call(
    _prep_body,
    out_shape=[
        jax.ShapeDtypeStruct((N, DEXT), jnp.float32),
        jax.ShapeDtypeStruct((N, 2), jnp.float32),
    ],
)


def _agg_body(emb_hbm, ar_hbm, ac_hbm, r_hbm, c_hbm, out_hbm,
              ar_t, ac_t, r_v, c_v, e_v, rows_v, acc, sem):
    cid = lax.axis_index("c")
    sid = lax.axis_index("s")
    wid = sid * NC + cid

    z16 = jnp.zeros((L,), jnp.float32)

    def zrow(i, carry):
        for ci in range(DEXT // L):
            rows_v[i, pl.ds(ci * L, L)] = z16
        return carry

    lax.fori_loop(0, K, zrow, 0)
    for z in range(NRCHUNK // NS + 1):
        c = z * NS + sid

        @pl.when(c < NRCHUNK)
        def _():
            pltpu.sync_copy(rows_v, acc.at[pl.ds(c * K, K)])

    pltpu.sync_copy(ar_hbm, ar_t)
    pltpu.sync_copy(ac_hbm, ac_t)
    plsc.subcore_barrier()

    base0 = wid * EPW

    def chunk(j, carry):
        base = base0 + j * K
        pltpu.sync_copy(r_hbm.at[pl.ds(base, K)], r_v)
        pltpu.sync_copy(c_hbm.at[pl.ds(base, K)], c_v)
        pltpu.async_copy(emb_hbm.at[c_v], rows_v, sem).wait()
        for i in range(K // L):
            r16 = r_v[pl.ds(i * L, L)]
            c16 = c_v[pl.ds(i * L, L)]
            s = plsc.load_gather(ar_t, [r16]) + plsc.load_gather(ac_t, [c16])
            e_v[pl.ds(i * L, L)] = jnp.exp(jnp.maximum(s, s * SLOPE))

        def sgrp(g, inner):
            e16 = e_v[pl.ds(g * L, L)]
            for t in range(L):
                ek = e16[t]
                k = g * L + t
                for ci in range(DEXT // L):
                    rows_v[k, pl.ds(ci * L, L)] = rows_v[k, pl.ds(ci * L, L)] * ek
            return inner

        lax.fori_loop(0, K // L, sgrp, 0)
        pltpu.sync_copy(rows_v, acc.at[r_v], add=True)
        return carry

    lax.fori_loop(0, NCHUNK, chunk, 0)

    plsc.subcore_barrier()
    for z in range(NRCHUNK // NS + 1):
        c = z * NS + sid

        @pl.when(c < NRCHUNK)
        def _():
            pltpu.sync_copy(acc.at[pl.ds(c * K, K)],
                            out_hbm.at[cid, pl.ds(c * K, K)])


_agg = pl.kernel(
    _agg_body,
    out_type=jax.ShapeDtypeStruct((NC, N, DEXT), jnp.float32),
    mesh=plsc.VectorSubcoreMesh(core_axis_name="c", subcore_axis_name="s"),
    compiler_params=pltpu.CompilerParams(
        needs_layout_passes=False, use_tc_tiling_on_sc=False),
    scratch_types=[
        pltpu.VMEM((N,), jnp.float32),
        pltpu.VMEM((N,), jnp.float32),
        pltpu.VMEM((K,), jnp.int32),
        pltpu.VMEM((K,), jnp.int32),
        pltpu.VMEM((K,), jnp.float32),
        pltpu.VMEM((K, DEXT), jnp.float32),
        pltpu.VMEM_SHARED((N, DEXT), jnp.float32),
        pltpu.SemaphoreType.DMA,
    ],
)


def _comb_body(p_ref, o_ref):
    num = p_ref[0, :, :D] + p_ref[1, :, :D]
    den = p_ref[0, :, D:D + 1] + p_ref[1, :, D:D + 1]
    o_ref[...] = num / (den + 1e-8)


_comb = pl.pallas_call(
    _comb_body,
    out_shape=jax.ShapeDtypeStruct((N, D), jnp.float32),
)


def kernel(features, W, b, a, edge_index, nodes, ind):
    a2d = jnp.concatenate([a[:D], a[D:]], axis=1)
    emb_ext, alphas = _prep(features, W, b.reshape(1, D), a2d)
    parts = _agg(emb_ext, alphas[:, 0], alphas[:, 1],
                 edge_index[0], edge_index[1])
    return _comb(parts)


# speedup vs baseline: 12.1057x; 1.9099x over previous
import jax
import jax.numpy as jnp
from jax import lax
from jax.experimental import pallas as pl
from jax.experimental.pallas import tpu as pltpu
from jax.experimental.pallas import tpu_sc as plsc

N = 10000
E = 320000
D = 128
DEXT = 144
CONE = D
CAC = D + 1
SLOPE = 0.1

NC, NS, L = 2, 16, 16
NW = NC * NS
EPW = E // NW
K = 80
NCHUNK = EPW // K
NRCHUNK = N // K


def _prep_body(f_ref, w_ref, b_ref, a_ref, emb_ref, al_ref):
    emb = jnp.dot(f_ref[...], w_ref[...], preferred_element_type=jnp.float32)
    emb = emb + b_ref[...]
    al = jnp.dot(emb, a_ref[...], preferred_element_type=jnp.float32)
    al_ref[...] = al
    ones = jnp.ones((N, 1), jnp.float32)
    pad = jnp.zeros((N, DEXT - D - 2), jnp.float32)
    emb_ref[...] = jnp.concatenate([emb, ones, al[:, 1:2], pad], axis=1)


_prep = pl.pallas_call(
    _prep_body,
    out_shape=[
        jax.ShapeDtypeStruct((N, DEXT), jnp.float32),
        jax.ShapeDtypeStruct((N, 2), jnp.float32),
    ],
)


def _agg_body(emb_hbm, ar_hbm, r_hbm, c_hbm, out_hbm,
              ar_t, rbuf, cbuf, rows0, rows1, acc,
              gsem0, gsem1, ssem0, ssem1, isem0, isem1, isem2, isem3):
    cid = lax.axis_index("c")
    sid = lax.axis_index("s")
    wid = sid * NC + cid
    rows = (rows0, rows1)
    gsem = (gsem0, gsem1)
    ssem = (ssem0, ssem1)
    isem = (isem0, isem1, isem2, isem3)

    z16 = jnp.zeros((L,), jnp.float32)

    def zrow(i, carry):
        for ci in range(DEXT // L):
            rows0[i, pl.ds(ci * L, L)] = z16
        return carry

    lax.fori_loop(0, K, zrow, 0)
    for z in range(NRCHUNK // NS + 1):
        c = z * NS + sid

        @pl.when(c < NRCHUNK)
        def _():
            pltpu.sync_copy(rows0, acc.at[pl.ds(c * K, K)])

    pltpu.sync_copy(ar_hbm, ar_t)
    plsc.subcore_barrier()

    lane = lax.iota(jnp.int32, L)
    col_ac = jnp.full((L,), CAC, jnp.int32)

    def idx_issue(j, q):
        pltpu.async_copy(r_hbm.at[wid, j], rbuf.at[q], isem[q])
        pltpu.async_copy(c_hbm.at[wid, j], cbuf.at[q], isem[q])

    def idx_wait(j, q):
        pltpu.make_async_copy(r_hbm.at[wid, j], rbuf.at[q], isem[q]).wait()
        pltpu.make_async_copy(c_hbm.at[wid, j], cbuf.at[q], isem[q]).wait()

    def gather(j, q, p):
        pltpu.async_copy(emb_hbm.at[cbuf.at[q]], rows[p], gsem[p])

    def gwait(j, q, p):
        pltpu.make_async_copy(emb_hbm.at[cbuf.at[q]], rows[p], gsem[p]).wait()

    def scatter(j, q, p):
        pltpu.async_copy(rows[p], acc.at[rbuf.at[q]], ssem[p], add=True)

    def swait(j, q, p):
        pltpu.make_async_copy(rows[p], acc.at[rbuf.at[q]], ssem[p]).wait()

    def scale(j, q, p):
        rp = rows[p]

        def grp(g, carry):
            r16 = rbuf[q, pl.ds(g * L, L)]
            ar16 = plsc.load_gather(ar_t, [r16])
            ac16 = plsc.load_gather(rp, [g * L + lane, col_ac])
            s = ar16 + ac16
            e16 = jnp.exp(jnp.maximum(s, s * SLOPE))
            for t in range(L):
                ek = e16[t]
                k = g * L + t
                for ci in range(DEXT // L):
                    rp[k, pl.ds(ci * L, L)] = rp[k, pl.ds(ci * L, L)] * ek
            return carry

        lax.fori_loop(0, K // L, grp, 0)

    idx_issue(0, 0)
    idx_issue(1, 1)
    idx_wait(0, 0)
    gather(0, 0, 0)

    def phase(j, m):
        p = m % 2
        idx_wait(j + 1, (m + 1) % 4)
        gwait(j, m, p)

        @pl.when(j > 0)
        def _():
            swait(j - 1, (m + 3) % 4, 1 - p)

        gather(j + 1, (m + 1) % 4, 1 - p)

        @pl.when(j + 2 < NCHUNK)
        def _():
            idx_issue(j + 2, (m + 2) % 4)

        scale(j, m, p)
        scatter(j, m, p)

    def quad(t, carry):
        for m in range(4):
            phase(4 * t + m, m)
        return carry

    lax.fori_loop(0, (NCHUNK - 1) // 4, quad, 0)

    jl = NCHUNK - 1
    gwait(jl, 0, 0)
    swait(jl - 1, 3, 1)
    scale(jl, 0, 0)
    scatter(jl, 0, 0)
    swait(jl, 0, 0)

    plsc.subcore_barrier()
    for z in range(NRCHUNK // NS + 1):
        c = z * NS + sid

        @pl.when(c < NRCHUNK)
        def _():
            pltpu.sync_copy(acc.at[pl.ds(c * K, K)],
                            out_hbm.at[cid, pl.ds(c * K, K)])


_agg = pl.kernel(
    _agg_body,
    out_type=jax.ShapeDtypeStruct((NC, N, DEXT), jnp.float32),
    mesh=plsc.VectorSubcoreMesh(core_axis_name="c", subcore_axis_name="s"),
    compiler_params=pltpu.CompilerParams(
        needs_layout_passes=False, use_tc_tiling_on_sc=False),
    scratch_types=[
        pltpu.VMEM((N,), jnp.float32),
        pltpu.VMEM((4, K), jnp.int32),
        pltpu.VMEM((4, K), jnp.int32),
        pltpu.VMEM((K, DEXT), jnp.float32),
        pltpu.VMEM((K, DEXT), jnp.float32),
        pltpu.VMEM_SHARED((N, DEXT), jnp.float32),
        pltpu.SemaphoreType.DMA,
        pltpu.SemaphoreType.DMA,
        pltpu.SemaphoreType.DMA,
        pltpu.SemaphoreType.DMA,
        pltpu.SemaphoreType.DMA,
        pltpu.SemaphoreType.DMA,
        pltpu.SemaphoreType.DMA,
        pltpu.SemaphoreType.DMA,
    ],
)


def _comb_body(p_ref, o_ref):
    num = p_ref[0, :, :D] + p_ref[1, :, :D]
    den = p_ref[0, :, CONE:CONE + 1] + p_ref[1, :, CONE:CONE + 1]
    o_ref[...] = num / (den + 1e-8)


_comb = pl.pallas_call(
    _comb_body,
    out_shape=jax.ShapeDtypeStruct((N, D), jnp.float32),
)


def kernel(features, W, b, a, edge_index, nodes, ind):
    a2d = jnp.concatenate([a[:D], a[D:]], axis=1)
    emb_ext, alphas = _prep(features, W, b.reshape(1, D), a2d)
    r3 = edge_index[0].reshape(NW, NCHUNK, K)
    c3 = edge_index[1].reshape(NW, NCHUNK, K)
    parts = _agg(emb_ext, alphas[:, 0], r3, c3)
    return _comb(parts)


# speedup vs baseline: 13.4599x; 1.1119x over previous
import jax
import jax.numpy as jnp
from jax import lax
from jax.experimental import pallas as pl
from jax.experimental.pallas import tpu as pltpu
from jax.experimental.pallas import tpu_sc as plsc

N = 10000
E = 320000
D = 128
DEXT = 144
CONE = D
CAC = D + 1
SLOPE = 0.1

NC, NS, L = 2, 16, 16
NW = NC * NS
EPW = E // NW
K = 80
NCHUNK = EPW // K
NRCHUNK = N // K
UNROLL = 6
NLOOP = 120


def _prep_body(f_ref, w_ref, b_ref, a_ref, emb_ref, al_ref):
    emb = jnp.dot(f_ref[...], w_ref[...], preferred_element_type=jnp.float32)
    emb = emb + b_ref[...]
    al = jnp.dot(emb, a_ref[...], preferred_element_type=jnp.float32)
    al_ref[...] = al
    ones = jnp.ones((N, 1), jnp.float32)
    pad = jnp.zeros((N, DEXT - D - 2), jnp.float32)
    emb_ref[...] = jnp.concatenate([emb, ones, al[:, 1:2], pad], axis=1)


_prep = pl.pallas_call(
    _prep_body,
    out_shape=[
        jax.ShapeDtypeStruct((N, DEXT), jnp.float32),
        jax.ShapeDtypeStruct((N, 2), jnp.float32),
    ],
)


def _agg_body(emb_hbm, ar_hbm, r_hbm, c_hbm, out_hbm,
              rbuf, cbuf, abuf, rows0, rows1, rows2, acc,
              gsem0, gsem1, gsem2, ssem0, ssem1, ssem2,
              isem0, isem1, isem2, isem3, isem4, isem5, asem0, asem1):
    cid = lax.axis_index("c")
    sid = lax.axis_index("s")
    wid = sid * NC + cid
    rows = (rows0, rows1, rows2)
    gsem = (gsem0, gsem1, gsem2)
    ssem = (ssem0, ssem1, ssem2)
    isem = (isem0, isem1, isem2, isem3, isem4, isem5)
    asem = (asem0, asem1)

    z16 = jnp.zeros((L,), jnp.float32)

    def zrow(i, carry):
        for ci in range(DEXT // L):
            rows0[i, pl.ds(ci * L, L)] = z16
        return carry

    lax.fori_loop(0, K, zrow, 0)
    for z in range(NRCHUNK // NS + 1):
        c = z * NS + sid

        @pl.when(c < NRCHUNK)
        def _():
            pltpu.sync_copy(rows0, acc.at[pl.ds(c * K, K)])

    plsc.subcore_barrier()

    def idx_issue(j, q):
        pltpu.async_copy(r_hbm.at[wid, j], rbuf.at[q], isem[q])
        pltpu.async_copy(c_hbm.at[wid, j], cbuf.at[q], isem[q])

    def idx_wait(j, q):
        pltpu.make_async_copy(r_hbm.at[wid, j], rbuf.at[q], isem[q]).wait()
        pltpu.make_async_copy(c_hbm.at[wid, j], cbuf.at[q], isem[q]).wait()

    def ar_issue(j, q, a):
        pltpu.async_copy(ar_hbm.at[rbuf.at[q]], abuf.at[a], asem[a])

    def ar_wait(j, q, a):
        pltpu.make_async_copy(ar_hbm.at[rbuf.at[q]], abuf.at[a], asem[a]).wait()

    def gather(j, q, p):
        pltpu.async_copy(emb_hbm.at[cbuf.at[q]], rows[p], gsem[p])

    def gwait(j, q, p):
        pltpu.make_async_copy(emb_hbm.at[cbuf.at[q]], rows[p], gsem[p]).wait()

    def scatter(j, q, p):
        pltpu.async_copy(rows[p], acc.at[rbuf.at[q]], ssem[p], add=True)

    def swait(j, q, p):
        pltpu.make_async_copy(rows[p], acc.at[rbuf.at[q]], ssem[p]).wait()

    lane = lax.iota(jnp.int32, L)
    col_ac = jnp.full((L,), CAC, jnp.int32)

    def scale(j, a, p):
        rp = rows[p]

        def grp(g, carry):
            ar16 = abuf[a, pl.ds(g * L, L)]
            ac16 = plsc.load_gather(rp, [g * L + lane, col_ac])
            s = ar16 + ac16
            e16 = jnp.exp(jnp.maximum(s, s * SLOPE))
            for t in range(L):
                ek = e16[t]
                k = g * L + t
                for ci in range(DEXT // L):
                    rp[k, pl.ds(ci * L, L)] = rp[k, pl.ds(ci * L, L)] * ek
            return carry

        lax.fori_loop(0, K // L, grp, 0)

    for jj in range(5):
        idx_issue(jj, jj)
    idx_wait(0, 0)
    ar_issue(0, 0, 0)
    gather(0, 0, 0)
    idx_wait(1, 1)
    gather(1, 1, 1)

    def phase(j, m, guard_swait, has_p1, has_p2, has_p5):
        p, q, a = m % 3, m % 6, m % 2
        if has_p1:
            ar_issue(j + 1, (q + 1) % 6, (a + 1) % 2)
        gwait(j, q, p)
        ar_wait(j, q, a)
        scale(j, a, p)
        scatter(j, q, p)

        def _wait_prev():
            swait(j - 1, (q + 5) % 6, (p + 2) % 3)

        if guard_swait:
            pl.when(j > 0)(_wait_prev)
        else:
            _wait_prev()
        if has_p2:
            idx_wait(j + 2, (q + 2) % 6)
            gather(j + 2, (q + 2) % 6, (p + 2) % 3)
        if has_p5:
            idx_issue(j + 5, (q + 5) % 6)

    def sixpack(t, carry):
        j0 = t * UNROLL
        for m in range(UNROLL):
            phase(j0 + m, m, m == 0, True, True, True)
        return carry

    lax.fori_loop(0, NLOOP // UNROLL, sixpack, 0)

    for j in range(NLOOP, NCHUNK):
        m = j % UNROLL
        phase(j, m, False, j + 1 < NCHUNK, j + 2 < NCHUNK, j + 5 < NCHUNK)
    swait(NCHUNK - 1, (NCHUNK - 1) % 6, (NCHUNK - 1) % 3)

    plsc.subcore_barrier()
    for z in range(NRCHUNK // NS + 1):
        c = z * NS + sid

        @pl.when(c < NRCHUNK)
        def _():
            pltpu.sync_copy(acc.at[pl.ds(c * K, K)],
                            out_hbm.at[cid, pl.ds(c * K, K)])


_agg = pl.kernel(
    _agg_body,
    out_type=jax.ShapeDtypeStruct((NC, N, DEXT), jnp.float32),
    mesh=plsc.VectorSubcoreMesh(core_axis_name="c", subcore_axis_name="s"),
    compiler_params=pltpu.CompilerParams(
        needs_layout_passes=False, use_tc_tiling_on_sc=False),
    scratch_types=[
        pltpu.VMEM((6, K), jnp.int32),
        pltpu.VMEM((6, K), jnp.int32),
        pltpu.VMEM((2, K), jnp.float32),
        pltpu.VMEM((K, DEXT), jnp.float32),
        pltpu.VMEM((K, DEXT), jnp.float32),
        pltpu.VMEM((K, DEXT), jnp.float32),
        pltpu.VMEM_SHARED((N, DEXT), jnp.float32),
        pltpu.SemaphoreType.DMA,
        pltpu.SemaphoreType.DMA,
        pltpu.SemaphoreType.DMA,
        pltpu.SemaphoreType.DMA,
        pltpu.SemaphoreType.DMA,
        pltpu.SemaphoreType.DMA,
        pltpu.SemaphoreType.DMA,
        pltpu.SemaphoreType.DMA,
        pltpu.SemaphoreType.DMA,
        pltpu.SemaphoreType.DMA,
        pltpu.SemaphoreType.DMA,
        pltpu.SemaphoreType.DMA,
        pltpu.SemaphoreType.DMA,
        pltpu.SemaphoreType.DMA,
    ],
)


def _comb_body(p_ref, o_ref):
    num = p_ref[0, :, :D] + p_ref[1, :, :D]
    den = p_ref[0, :, CONE:CONE + 1] + p_ref[1, :, CONE:CONE + 1]
    o_ref[...] = num / (den + 1e-8)


_comb = pl.pallas_call(
    _comb_body,
    out_shape=jax.ShapeDtypeStruct((N, D), jnp.float32),
)


def kernel(features, W, b, a, edge_index, nodes, ind):
    a2d = jnp.concatenate([a[:D], a[D:]], axis=1)
    emb_ext, alphas = _prep(features, W, b.reshape(1, D), a2d)
    r3 = edge_index[0].reshape(NW, NCHUNK, K)
    c3 = edge_index[1].reshape(NW, NCHUNK, K)
    parts = _agg(emb_ext, alphas[:, 0], r3, c3)
    return _comb(parts)


# speedup vs baseline: 14.8520x; 1.1034x over previous
import jax
import jax.numpy as jnp
from jax import lax
from jax.experimental import pallas as pl
from jax.experimental.pallas import tpu as pltpu
from jax.experimental.pallas import tpu_sc as plsc

N = 10000
E = 320000
D = 128
SLOPE = 0.1

NC, NS, L = 2, 16, 16
NW = NC * NS
EPW = E // NW
K = 80
NCHUNK = EPW // K
NRCHUNK = N // K
UNROLL = 6
NLOOP = 120


def _prep_body(f_ref, w_ref, b_ref, a_ref, emb_ref, al_ref):
    emb = jnp.dot(f_ref[...], w_ref[...], preferred_element_type=jnp.float32)
    emb = emb + b_ref[...]
    al_ref[...] = jnp.dot(emb, a_ref[...], preferred_element_type=jnp.float32)
    emb_ref[...] = emb


_prep = pl.pallas_call(
    _prep_body,
    out_shape=[
        jax.ShapeDtypeStruct((N, D), jnp.float32),
        jax.ShapeDtypeStruct((N, 2), jnp.float32),
    ],
)


def _agg_body(emb_hbm, ar_hbm, ac_hbm, r_hbm, c_hbm, num_hbm, den_hbm,
              rbuf, cbuf, abuf, bbuf, ebuf, rows0, rows1, rows2, acc, dacc,
              gsem0, gsem1, gsem2, ssem0, ssem1, ssem2,
              isem0, isem1, isem2, isem3, isem4, isem5,
              asem0, asem1, esem0, esem1):
    cid = lax.axis_index("c")
    sid = lax.axis_index("s")
    wid = sid * NC + cid
    rows = (rows0, rows1, rows2)
    gsem = (gsem0, gsem1, gsem2)
    ssem = (ssem0, ssem1, ssem2)
    isem = (isem0, isem1, isem2, isem3, isem4, isem5)
    asem = (asem0, asem1)
    esem = (esem0, esem1)

    z16 = jnp.zeros((L,), jnp.float32)

    def zrow(i, carry):
        for ci in range(D // L):
            rows0[i, pl.ds(ci * L, L)] = z16
        return carry

    lax.fori_loop(0, K, zrow, 0)
    for z in range(NRCHUNK // NS + 1):
        c = z * NS + sid

        @pl.when(c < NRCHUNK)
        def _():
            pltpu.sync_copy(rows0, acc.at[pl.ds(c * K, K)])
            pltpu.sync_copy(rows0.at[0, pl.ds(0, K)], dacc.at[pl.ds(c * K, K)])

    plsc.subcore_barrier()

    def idx_issue(j, q):
        pltpu.async_copy(r_hbm.at[wid, j], rbuf.at[q], isem[q])
        pltpu.async_copy(c_hbm.at[wid, j], cbuf.at[q], isem[q])

    def idx_wait(j, q):
        pltpu.make_async_copy(r_hbm.at[wid, j], rbuf.at[q], isem[q]).wait()
        pltpu.make_async_copy(c_hbm.at[wid, j], cbuf.at[q], isem[q]).wait()

    def ar_issue(j, q, a):
        pltpu.async_copy(ar_hbm.at[rbuf.at[q]], abuf.at[a], asem[a])
        pltpu.async_copy(ac_hbm.at[cbuf.at[q]], bbuf.at[a], asem[a])

    def ar_wait(j, q, a):
        pltpu.make_async_copy(ar_hbm.at[rbuf.at[q]], abuf.at[a], asem[a]).wait()
        pltpu.make_async_copy(ac_hbm.at[cbuf.at[q]], bbuf.at[a], asem[a]).wait()

    def gather(j, q, p):
        pltpu.async_copy(emb_hbm.at[cbuf.at[q]], rows[p], gsem[p])

    def gwait(j, q, p):
        pltpu.make_async_copy(emb_hbm.at[cbuf.at[q]], rows[p], gsem[p]).wait()

    def scatter(j, q, p, a):
        pltpu.async_copy(rows[p], acc.at[rbuf.at[q]], ssem[p], add=True)
        pltpu.async_copy(ebuf.at[a], dacc.at[rbuf.at[q]], esem[a], add=True)

    def swait(j, q, p, a):
        pltpu.make_async_copy(rows[p], acc.at[rbuf.at[q]], ssem[p]).wait()
        pltpu.make_async_copy(ebuf.at[a], dacc.at[rbuf.at[q]], esem[a]).wait()

    def scale(j, a, p):
        rp = rows[p]

        def grp(g, carry):
            s = abuf[a, pl.ds(g * L, L)] + bbuf[a, pl.ds(g * L, L)]
            e16 = jnp.exp(jnp.maximum(s, s * SLOPE))
            ebuf[a, pl.ds(g * L, L)] = e16
            for t in range(L):
                ek = e16[t]
                k = g * L + t
                for ci in range(D // L):
                    rp[k, pl.ds(ci * L, L)] = rp[k, pl.ds(ci * L, L)] * ek
            return carry

        lax.fori_loop(0, K // L, grp, 0)

    for jj in range(5):
        idx_issue(jj, jj)
    idx_wait(0, 0)
    ar_issue(0, 0, 0)
    gather(0, 0, 0)
    idx_wait(1, 1)
    gather(1, 1, 1)

    def phase(j, m, guard_swait, has_p1, has_p2, has_p5):
        p, q, a = m % 3, m % 6, m % 2
        if has_p1:
            ar_issue(j + 1, (q + 1) % 6, (a + 1) % 2)
        gwait(j, q, p)
        ar_wait(j, q, a)
        scale(j, a, p)
        scatter(j, q, p, a)

        def _wait_prev():
            swait(j - 1, (q + 5) % 6, (p + 2) % 3, (a + 1) % 2)

        if guard_swait:
            pl.when(j > 0)(_wait_prev)
        else:
            _wait_prev()
        if has_p2:
            idx_wait(j + 2, (q + 2) % 6)
            gather(j + 2, (q + 2) % 6, (p + 2) % 3)
        if has_p5:
            idx_issue(j + 5, (q + 5) % 6)

    def sixpack(t, carry):
        j0 = t * UNROLL
        for m in range(UNROLL):
            phase(j0 + m, m, m == 0, True, True, True)
        return carry

    lax.fori_loop(0, NLOOP // UNROLL, sixpack, 0)

    for j in range(NLOOP, NCHUNK):
        m = j % UNROLL
        phase(j, m, False, j + 1 < NCHUNK, j + 2 < NCHUNK, j + 5 < NCHUNK)
    swait(NCHUNK - 1, (NCHUNK - 1) % 6, (NCHUNK - 1) % 3, (NCHUNK - 1) % 2)

    plsc.subcore_barrier()
    for z in range(NRCHUNK // NS + 1):
        c = z * NS + sid

        @pl.when(c < NRCHUNK)
        def _():
            pltpu.sync_copy(acc.at[pl.ds(c * K, K)],
                            num_hbm.at[cid, pl.ds(c * K, K)])
            pltpu.sync_copy(dacc.at[pl.ds(c * K, K)],
                            den_hbm.at[cid, pl.ds(c * K, K)])


_agg = pl.kernel(
    _agg_body,
    out_type=[
        jax.ShapeDtypeStruct((NC, N, D), jnp.float32),
        jax.ShapeDtypeStruct((NC, N), jnp.float32),
    ],
    mesh=plsc.VectorSubcoreMesh(core_axis_name="c", subcore_axis_name="s"),
    compiler_params=pltpu.CompilerParams(
        needs_layout_passes=False, use_tc_tiling_on_sc=False),
    scratch_types=[
        pltpu.VMEM((6, K), jnp.int32),
        pltpu.VMEM((6, K), jnp.int32),
        pltpu.VMEM((2, K), jnp.float32),
        pltpu.VMEM((2, K), jnp.float32),
        pltpu.VMEM((2, K), jnp.float32),
        pltpu.VMEM((K, D), jnp.float32),
        pltpu.VMEM((K, D), jnp.float32),
        pltpu.VMEM((K, D), jnp.float32),
        pltpu.VMEM_SHARED((N, D), jnp.float32),
        pltpu.VMEM_SHARED((N,), jnp.float32),
        pltpu.SemaphoreType.DMA,
        pltpu.SemaphoreType.DMA,
        pltpu.SemaphoreType.DMA,
        pltpu.SemaphoreType.DMA,
        pltpu.SemaphoreType.DMA,
        pltpu.SemaphoreType.DMA,
        pltpu.SemaphoreType.DMA,
        pltpu.SemaphoreType.DMA,
        pltpu.SemaphoreType.DMA,
        pltpu.SemaphoreType.DMA,
        pltpu.SemaphoreType.DMA,
        pltpu.SemaphoreType.DMA,
        pltpu.SemaphoreType.DMA,
        pltpu.SemaphoreType.DMA,
        pltpu.SemaphoreType.DMA,
        pltpu.SemaphoreType.DMA,
    ],
)


def _comb_body(p_ref, d_ref, o_ref):
    num = p_ref[0] + p_ref[1]
    den = d_ref[0] + d_ref[1]
    o_ref[...] = num / (den + 1e-8)


_comb = pl.pallas_call(
    _comb_body,
    out_shape=jax.ShapeDtypeStruct((N, D), jnp.float32),
)


def kernel(features, W, b, a, edge_index, nodes, ind):
    a2d = jnp.concatenate([a[:D], a[D:]], axis=1)
    emb, alphas = _prep(features, W, b.reshape(1, D), a2d)
    r3 = edge_index[0].reshape(NW, NCHUNK, K)
    c3 = edge_index[1].reshape(NW, NCHUNK, K)
    num, den = _agg(emb, alphas[:, 0], alphas[:, 1], r3, c3)
    return _comb(num, den.reshape(NC, N, 1))
